# baseline (speedup 1.0000x reference)
import math

import jax
import jax.numpy as jnp
from jax.experimental import pallas as pl
from jax.experimental.pallas import tpu as pltpu

_E = 8
_N_TOK = 8192
_TB = _N_TOK // _E
_D = 2048
_H = _D // 2
_CLAMP_MAX = math.log(100.0)


def _fused_body(t1_ref, t2_ref, x_hbm, w1_hbm, b1_ref, w2_ref, b2_ref,
                o_ref, xbuf, w1buf, h_ref, semx, semw):
    e = pl.program_id(0)
    ph = pl.program_id(1)

    @pl.when((ph == 0) & (e == 0))
    def _first_fetch():
        cpx = pltpu.make_async_copy(x_hbm.at[pl.ds(0, _TB), :], xbuf, semx)
        cpw = pltpu.make_async_copy(w1_hbm.at[0], w1buf, semw)
        cpx.start()
        cpw.start()
        cpx.wait()
        cpw.wait()

    @pl.when((ph == 0) & (e > 0))
    def _await_fetch():
        pltpu.make_async_copy(
            x_hbm.at[pl.ds(e * _TB, _TB), :], xbuf, semx).wait()
        pltpu.make_async_copy(w1_hbm.at[e], w1buf, semw).wait()

    @pl.when((ph == 1) & (e < _E - 1))
    def _prefetch_next():
        pltpu.make_async_copy(
            x_hbm.at[pl.ds((e + 1) * _TB, _TB), :], xbuf, semx).start()
        pltpu.make_async_copy(w1_hbm.at[e + 1], w1buf, semw).start()

    @pl.when(ph == 0)
    def _layer1():
        hq = jnp.dot(xbuf[...], w1buf[...],
                     preferred_element_type=jnp.float32)
        h_ref[...] = jnp.maximum(hq + b1_ref[0], 0.0).astype(jnp.bfloat16)

    @pl.when(ph >= 1)
    def _layer2():
        s1 = jnp.exp(jnp.minimum(t1_ref[0], _CLAMP_MAX))
        s2 = jnp.exp(jnp.minimum(t2_ref[0], _CLAMP_MAX))
        acc = jnp.dot(h_ref[...], w2_ref[0],
                      preferred_element_type=jnp.float32)
        o_ref[...] = acc * (s1 * s2) + b2_ref[0] * s2


def _w2_index(e, ph):
    ec = jnp.where(ph == 0, jnp.maximum(e - 1, 0), e)
    j = jnp.where(ph == 0, 1, ph - 1)
    return (ec, 0, j)


def kernel(x, W1, b1, W2, b2, t1, t2):
    b1r = b1.reshape(_E, 1, _D)
    b2r = b2.reshape(_E, 1, _D)
    grid = (_E, 3)
    return pl.pallas_call(
        _fused_body,
        grid=grid,
        in_specs=[
            pl.BlockSpec(memory_space=pltpu.SMEM),
            pl.BlockSpec(memory_space=pltpu.SMEM),
            pl.BlockSpec(memory_space=pl.ANY),
            pl.BlockSpec(memory_space=pl.ANY),
            pl.BlockSpec((1, 1, _D), lambda e, ph: (e, 0, 0)),
            pl.BlockSpec((1, _D, _H), _w2_index),
            pl.BlockSpec((1, 1, _H),
                         lambda e, ph: (e, 0, jnp.maximum(ph - 1, 0))),
        ],
        out_specs=pl.BlockSpec(
            (_TB, _H), lambda e, ph: (e, jnp.maximum(ph - 1, 0))
        ),
        out_shape=jax.ShapeDtypeStruct((_N_TOK, _D), jnp.float32),
        scratch_shapes=[
            pltpu.VMEM((_TB, _D), jnp.float32),
            pltpu.VMEM((_D, _D), jnp.float32),
            pltpu.VMEM((_TB, _D), jnp.bfloat16),
            pltpu.SemaphoreType.DMA,
            pltpu.SemaphoreType.DMA,
        ],
        compiler_params=pltpu.CompilerParams(
            dimension_semantics=("arbitrary", "arbitrary"),
        ),
    )(t1, t2, x, W1, b1r, W2, b2r)


# speedup vs baseline: 1.1375x; 1.1375x over previous
import math

import jax
import jax.numpy as jnp
from jax.experimental import pallas as pl
from jax.experimental.pallas import tpu as pltpu

_E = 8
_N_TOK = 8192
_TB = _N_TOK // _E
_D = 2048
_H = _D // 2
_CLAMP_MAX = math.log(100.0)


def _fused_body(t1_ref, t2_ref, x_hbm, w1_ref, b1_ref, w2_ref, b2_ref,
                o_ref, xbuf, h_ref, sem):
    e = pl.program_id(0)
    ph = pl.program_id(1)

    @pl.when((ph == 0) & (e == 0))
    def _first_fetch():
        cp = pltpu.make_async_copy(x_hbm.at[pl.ds(0, _TB), :], xbuf, sem)
        cp.start()
        cp.wait()

    @pl.when((ph == 0) & (e > 0))
    def _await_fetch():
        pltpu.make_async_copy(
            x_hbm.at[pl.ds(e * _TB, _TB), :], xbuf, sem).wait()

    @pl.when((ph == 2) & (e < _E - 1))
    def _prefetch_next():
        pltpu.make_async_copy(
            x_hbm.at[pl.ds((e + 1) * _TB, _TB), :], xbuf, sem).start()

    @pl.when(ph < 2)
    def _layer1():
        hq = jnp.dot(xbuf[...], w1_ref[0], preferred_element_type=jnp.float32)
        h_ref[ph] = jnp.maximum(hq + b1_ref[0], 0.0).astype(jnp.bfloat16)

    @pl.when(ph >= 2)
    def _layer2():
        s1 = jnp.exp(jnp.minimum(t1_ref[0], _CLAMP_MAX))
        s2 = jnp.exp(jnp.minimum(t2_ref[0], _CLAMP_MAX))
        acc = jnp.dot(h_ref[0], w2_ref[0, 0:_H],
                      preferred_element_type=jnp.float32)
        acc = acc + jnp.dot(h_ref[1], w2_ref[0, _H:_D],
                            preferred_element_type=jnp.float32)
        o_ref[...] = acc * (s1 * s2) + b2_ref[0] * s2


def _w2_index(e, ph):
    ec = jnp.where(ph < 2, jnp.maximum(e - 1, 0), e)
    j = jnp.where(ph < 2, 1, ph - 2)
    return (ec, 0, j)


def kernel(x, W1, b1, W2, b2, t1, t2):
    b1r = b1.reshape(_E, 1, _D)
    b2r = b2.reshape(_E, 1, _D)
    grid = (_E, 4)
    return pl.pallas_call(
        _fused_body,
        grid=grid,
        in_specs=[
            pl.BlockSpec(memory_space=pltpu.SMEM),
            pl.BlockSpec(memory_space=pltpu.SMEM),
            pl.BlockSpec(memory_space=pl.ANY),
            pl.BlockSpec((1, _D, _H),
                         lambda e, ph: (e, 0, jnp.minimum(ph, 1))),
            pl.BlockSpec((1, 1, _H),
                         lambda e, ph: (e, 0, jnp.minimum(ph, 1))),
            pl.BlockSpec((1, _D, _H), _w2_index),
            pl.BlockSpec((1, 1, _H),
                         lambda e, ph: (e, 0, jnp.maximum(ph - 2, 0))),
        ],
        out_specs=pl.BlockSpec(
            (_TB, _H), lambda e, ph: (e, jnp.maximum(ph - 2, 0))
        ),
        out_shape=jax.ShapeDtypeStruct((_N_TOK, _D), jnp.float32),
        scratch_shapes=[
            pltpu.VMEM((_TB, _D), jnp.float32),
            pltpu.VMEM((2, _TB, _H), jnp.bfloat16),
            pltpu.SemaphoreType.DMA,
        ],
        compiler_params=pltpu.CompilerParams(
            dimension_semantics=("arbitrary", "arbitrary"),
        ),
    )(t1, t2, x, W1, b1r, W2, b2r)
